# baseline (speedup 1.0000x reference)
import functools

import jax
import jax.numpy as jnp
from jax.experimental import pallas as pl

_HI = jax.lax.Precision.HIGHEST


def _stage1_kernel(ent_ref, ne_ref, nr_ref, w1a_ref, wb2_ref, wc2_ref,
                   s2_ref, b1_ref, b2_ref, wg_ref, y_ref, *, K):
    ent = ent_ref[...]
    e1 = jnp.dot(ent, w1a_ref[...], precision=_HI) + b1_ref[...]
    e1t = jnp.concatenate([e1, e1], axis=1)
    wb2 = wb2_ref[...]
    wc2 = wc2_ref[...]
    s2 = s2_ref[...]
    npair = K // 2
    logit_cols = []
    for j in range(npair):
        ne_p = ne_ref[:, 128 * j:128 * (j + 1)]
        nr_p = nr_ref[:, 128 * j:128 * (j + 1)]
        h = jnp.dot(ne_p, wb2, precision=_HI)
        h = h + jnp.dot(nr_p, wc2, precision=_HI)
        h = jax.nn.relu(h + e1t)
        logit_cols.append(jnp.dot(h, s2, precision=_HI))
    logits = jnp.concatenate(logit_cols, axis=1) + b2_ref[0, 0]
    att = jax.nn.softmax(logits, axis=-1)
    agg2 = jnp.zeros_like(e1t)
    for j in range(npair):
        a0 = jax.lax.broadcast_in_dim(att[:, 2 * j], agg2.shape[:1] + (64,), (0,))
        a1 = jax.lax.broadcast_in_dim(att[:, 2 * j + 1], agg2.shape[:1] + (64,), (0,))
        a2 = jnp.concatenate([a0, a1], axis=1)
        agg2 = agg2 + a2 * ne_ref[:, 128 * j:128 * (j + 1)]
    agg = agg2[:, :64] + agg2[:, 64:]
    ea = jnp.concatenate([ent, agg], axis=1)
    y_ref[...] = jnp.dot(ea, wg_ref[...], precision=_HI)


def _stage2_kernel(y_ref, ka_ref, bgt_ref, g3t_ref, b3t_ref, gsel_ref,
                   wa_ref, ba_ref, q_ref, g4_ref, b4_ref, out_ref, *, N, D):
    y = y_ref[...]
    gx = jnp.dot(y, ka_ref[...], precision=_HI)
    pre = jnp.tanh(gx + bgt_ref[...])
    gsel = gsel_ref[...]
    mu = jnp.dot(pre, gsel, precision=_HI) * (1.0 / D)
    mub = jnp.dot(mu, gsel.T, precision=_HI)
    xc = pre - mub
    var = jnp.dot(xc * xc, gsel, precision=_HI) * (1.0 / D)
    varb = jnp.dot(var, gsel.T, precision=_HI)
    ei = xc * jax.lax.rsqrt(varb + 1e-5) * g3t_ref[...] + b3t_ref[...]
    wa = wa_ref[...]
    ba = ba_ref[...]
    qv = q_ref[...]
    lcols = []
    for n in range(N):
        ein = ei[:, D * n:D * (n + 1)]
        t = jnp.tanh(jnp.dot(ein, wa, precision=_HI) + ba)
        lcols.append(jnp.dot(t, qv, precision=_HI))
    wl = jnp.concatenate(lcols, axis=1)
    w = jax.nn.softmax(wl, axis=-1)
    target = jnp.zeros_like(ei[:, :D])
    for n in range(N):
        wn = jax.lax.broadcast_in_dim(w[:, n], target.shape, (0,))
        target = target + wn * ei[:, D * n:D * (n + 1)]
    mu2 = jnp.mean(target, axis=1, keepdims=True)
    xc2 = target - mu2
    var2 = jnp.mean(xc2 * xc2, axis=1, keepdims=True)
    out_ref[...] = xc2 * jax.lax.rsqrt(var2 + 1e-5) * g4_ref[...] + b4_ref[...]


@jax.jit
def kernel(entity_embedding, neigh_entity_embedding, neigh_relation_embedding,
           W1, b1, W2, b2, A, Wg, bg, g3, b3, Wa, ba, q, g4, b4):
    B, N, K, d = neigh_entity_embedding.shape
    att_dim = Wg.shape[1]
    qd = Wa.shape[1]
    f32 = jnp.float32

    ent2 = entity_embedding.reshape(B * N, d)
    ne4 = neigh_entity_embedding.reshape(B * N, K * d)
    nr4 = neigh_relation_embedding.reshape(B * N, K * d)
    W1a, W1b, W1c = W1[:d], W1[d:2 * d], W1[2 * d:]
    z = jnp.zeros((d, att_dim), f32)
    wb2 = jnp.block([[W1b, z], [z, W1b]])
    wc2 = jnp.block([[W1c, z], [z, W1c]])
    z1 = jnp.zeros((att_dim, 1), f32)
    s2 = jnp.block([[W2, z1], [z1, W2]])
    b1r = b1.reshape(1, att_dim)
    b2r = b2.reshape(1, 1)

    BB1 = 64
    m2 = BB1 * N
    grid1 = B // BB1
    y = pl.pallas_call(
        functools.partial(_stage1_kernel, K=K),
        grid=(grid1,),
        in_specs=[
            pl.BlockSpec((m2, d), lambda i: (i, 0)),
            pl.BlockSpec((m2, K * d), lambda i: (i, 0)),
            pl.BlockSpec((m2, K * d), lambda i: (i, 0)),
            pl.BlockSpec((d, att_dim), lambda i: (0, 0)),
            pl.BlockSpec((2 * d, 2 * att_dim), lambda i: (0, 0)),
            pl.BlockSpec((2 * d, 2 * att_dim), lambda i: (0, 0)),
            pl.BlockSpec((2 * att_dim, 2), lambda i: (0, 0)),
            pl.BlockSpec((1, att_dim), lambda i: (0, 0)),
            pl.BlockSpec((1, 1), lambda i: (0, 0)),
            pl.BlockSpec((2 * d, att_dim), lambda i: (0, 0)),
        ],
        out_specs=pl.BlockSpec((m2, att_dim), lambda i: (i, 0)),
        out_shape=jax.ShapeDtypeStruct((B * N, att_dim), f32),
    )(ent2, ne4, nr4, W1a, wb2, wc2, s2, b1r, b2r, Wg)

    yg = y.reshape(B, N * att_dim)
    ka = jnp.kron(A.T, jnp.eye(att_dim, dtype=f32))
    gsel = jnp.kron(jnp.eye(N, dtype=f32), jnp.ones((att_dim, 1), f32))
    bgt = jnp.tile(bg, N).reshape(1, N * att_dim)
    g3t = jnp.tile(g3, N).reshape(1, N * att_dim)
    b3t = jnp.tile(b3, N).reshape(1, N * att_dim)
    bar = ba.reshape(1, qd)
    qr = q.reshape(qd, 1)
    g4r = g4.reshape(1, att_dim)
    b4r = b4.reshape(1, att_dim)

    BB2 = 256
    grid2 = B // BB2
    out = pl.pallas_call(
        functools.partial(_stage2_kernel, N=N, D=att_dim),
        grid=(grid2,),
        in_specs=[
            pl.BlockSpec((BB2, N * att_dim), lambda i: (i, 0)),
            pl.BlockSpec((N * att_dim, N * att_dim), lambda i: (0, 0)),
            pl.BlockSpec((1, N * att_dim), lambda i: (0, 0)),
            pl.BlockSpec((1, N * att_dim), lambda i: (0, 0)),
            pl.BlockSpec((1, N * att_dim), lambda i: (0, 0)),
            pl.BlockSpec((N * att_dim, N), lambda i: (0, 0)),
            pl.BlockSpec((att_dim, qd), lambda i: (0, 0)),
            pl.BlockSpec((1, qd), lambda i: (0, 0)),
            pl.BlockSpec((qd, 1), lambda i: (0, 0)),
            pl.BlockSpec((1, att_dim), lambda i: (0, 0)),
            pl.BlockSpec((1, att_dim), lambda i: (0, 0)),
        ],
        out_specs=pl.BlockSpec((BB2, att_dim), lambda i: (i, 0)),
        out_shape=jax.ShapeDtypeStruct((B, att_dim), f32),
    )(yg, ka, bgt, g3t, b3t, gsel, Wa, bar, qr, g4r, b4r)
    return out


# speedup vs baseline: 1.8066x; 1.8066x over previous
import functools

import jax
import jax.numpy as jnp
from jax.experimental import pallas as pl
from jax.experimental.pallas import tpu as pltpu


def _copies(ne_hbm, nr_hbm, bufne, bufnr, sem, step, slot, *, BB, N, K, d):
    ib = step // N
    nn = step % N
    out = []
    for k in range(K):
        out.append(pltpu.make_async_copy(
            ne_hbm.at[pl.ds(ib * BB, BB), nn, k, :],
            bufne.at[slot, pl.ds(k * BB, BB), :],
            sem.at[slot],
        ))
        out.append(pltpu.make_async_copy(
            nr_hbm.at[pl.ds(ib * BB, BB), nn, k, :],
            bufnr.at[slot, pl.ds(k * BB, BB), :],
            sem.at[slot],
        ))
    return out


def _issue_copies(*a, **kw):
    for c in _copies(*a, **kw):
        c.start()


def _wait_copies(*a, **kw):
    for c in _copies(*a, **kw):
        c.wait()


def _stage1_kernel(ent_ref, ne_hbm, nr_hbm, w1a_ref, w1bc_ref, w2_ref,
                   b1_ref, wg_ref, y_ref, bufne, bufnr, sem, *, BB, N, K, d, steps):
    i = pl.program_id(0)
    n = pl.program_id(1)
    s = i * N + n
    slot = jax.lax.rem(s, 2)
    args = (ne_hbm, nr_hbm, bufne, bufnr, sem)
    kw = dict(BB=BB, N=N, K=K, d=d)

    @pl.when(s == 0)
    def _prologue():
        _issue_copies(*args, 0, 0, **kw)

    @pl.when(s + 1 < steps)
    def _prefetch():
        _issue_copies(*args, s + 1, 1 - slot, **kw)

    _wait_copies(*args, s, slot, **kw)

    entn = ent_ref[...]
    e1 = jnp.dot(entn, w1a_ref[...]) + b1_ref[...]
    e1t = jnp.concatenate([e1] * K, axis=0)
    nev = bufne[slot]
    nrv = bufnr[slot]
    cat = jnp.concatenate([nev, nrv], axis=1)
    h = jax.nn.relu(jnp.dot(cat, w1bc_ref[...]) + e1t)
    lcol = jnp.dot(h, w2_ref[...])
    lkb = lcol.reshape(K, BB)
    m = jnp.max(lkb, axis=0, keepdims=True)
    p = jnp.exp(lkb - m)
    att = p / jnp.sum(p, axis=0, keepdims=True)
    attc = att.reshape(K * BB, 1)
    w = jax.lax.broadcast_in_dim(attc[:, 0], (K * BB, d), (0,))
    wne = w * nev
    agg = jnp.sum(wne.reshape(K, BB, d), axis=0)
    ea = jnp.concatenate([entn, agg], axis=1)
    y_ref[...] = jnp.dot(ea, wg_ref[...])


def _stage2_kernel(y_ref, ka_ref, bgt_ref, g3t_ref, b3t_ref, gsel_ref,
                   wa_ref, ba_ref, q_ref, g4_ref, b4_ref, out_ref, *, N, D):
    y = y_ref[...]
    gx = jnp.dot(y, ka_ref[...])
    pre = jnp.tanh(gx + bgt_ref[...])
    gsel = gsel_ref[...]
    mu = jnp.dot(pre, gsel) * (1.0 / D)
    mub = jnp.dot(mu, gsel.T)
    xc = pre - mub
    var = jnp.dot(xc * xc, gsel) * (1.0 / D)
    varb = jnp.dot(var, gsel.T)
    ei = xc * jax.lax.rsqrt(varb + 1e-5) * g3t_ref[...] + b3t_ref[...]
    wa = wa_ref[...]
    ba = ba_ref[...]
    qv = q_ref[...]
    lcols = []
    for n in range(N):
        ein = ei[:, D * n:D * (n + 1)]
        t = jnp.tanh(jnp.dot(ein, wa) + ba)
        lcols.append(jnp.dot(t, qv))
    wl = jnp.concatenate(lcols, axis=1)
    w = jax.nn.softmax(wl, axis=-1)
    target = jnp.zeros_like(ei[:, :D])
    for n in range(N):
        wn = jax.lax.broadcast_in_dim(w[:, n], target.shape, (0,))
        target = target + wn * ei[:, D * n:D * (n + 1)]
    mu2 = jnp.mean(target, axis=1, keepdims=True)
    xc2 = target - mu2
    var2 = jnp.mean(xc2 * xc2, axis=1, keepdims=True)
    out_ref[...] = xc2 * jax.lax.rsqrt(var2 + 1e-5) * g4_ref[...] + b4_ref[...]


@jax.jit
def kernel(entity_embedding, neigh_entity_embedding, neigh_relation_embedding,
           W1, b1, W2, b2, A, Wg, bg, g3, b3, Wa, ba, q, g4, b4):
    B, N, K, d = neigh_entity_embedding.shape
    att_dim = Wg.shape[1]
    qd = Wa.shape[1]
    f32 = jnp.float32

    W1a = W1[:d]
    w1bc = W1[d:]
    b1r = b1.reshape(1, att_dim)
    ent_t = jnp.transpose(entity_embedding, (1, 0, 2))

    BB1 = 512
    steps1 = (B // BB1) * N
    y3 = pl.pallas_call(
        functools.partial(_stage1_kernel, BB=BB1, N=N, K=K, d=d, steps=steps1),
        grid=(B // BB1, N),
        in_specs=[
            pl.BlockSpec((None, BB1, d), lambda i, n: (n, i, 0)),
            pl.BlockSpec(memory_space=pltpu.MemorySpace.HBM),
            pl.BlockSpec(memory_space=pltpu.MemorySpace.HBM),
            pl.BlockSpec((d, att_dim), lambda i, n: (0, 0)),
            pl.BlockSpec((2 * d, att_dim), lambda i, n: (0, 0)),
            pl.BlockSpec((att_dim, 1), lambda i, n: (0, 0)),
            pl.BlockSpec((1, att_dim), lambda i, n: (0, 0)),
            pl.BlockSpec((2 * d, att_dim), lambda i, n: (0, 0)),
        ],
        out_specs=pl.BlockSpec((None, BB1, att_dim), lambda i, n: (n, i, 0)),
        out_shape=jax.ShapeDtypeStruct((N, B, att_dim), f32),
        scratch_shapes=[
            pltpu.VMEM((2, K * BB1, d), f32),
            pltpu.VMEM((2, K * BB1, d), f32),
            pltpu.SemaphoreType.DMA((2,)),
        ],
    )(ent_t, neigh_entity_embedding, neigh_relation_embedding,
      W1a, w1bc, W2, b1r, Wg)

    yg = jnp.transpose(y3, (1, 0, 2)).reshape(B, N * att_dim)
    ka = jnp.kron(A.T, jnp.eye(att_dim, dtype=f32))
    gsel = jnp.kron(jnp.eye(N, dtype=f32), jnp.ones((att_dim, 1), f32))
    bgt = jnp.tile(bg, N).reshape(1, N * att_dim)
    g3t = jnp.tile(g3, N).reshape(1, N * att_dim)
    b3t = jnp.tile(b3, N).reshape(1, N * att_dim)
    bar = ba.reshape(1, qd)
    qr = q.reshape(qd, 1)
    g4r = g4.reshape(1, att_dim)
    b4r = b4.reshape(1, att_dim)

    BB2 = 256
    grid2 = B // BB2
    out = pl.pallas_call(
        functools.partial(_stage2_kernel, N=N, D=att_dim),
        grid=(grid2,),
        in_specs=[
            pl.BlockSpec((BB2, N * att_dim), lambda i: (i, 0)),
            pl.BlockSpec((N * att_dim, N * att_dim), lambda i: (0, 0)),
            pl.BlockSpec((1, N * att_dim), lambda i: (0, 0)),
            pl.BlockSpec((1, N * att_dim), lambda i: (0, 0)),
            pl.BlockSpec((1, N * att_dim), lambda i: (0, 0)),
            pl.BlockSpec((N * att_dim, N), lambda i: (0, 0)),
            pl.BlockSpec((att_dim, qd), lambda i: (0, 0)),
            pl.BlockSpec((1, qd), lambda i: (0, 0)),
            pl.BlockSpec((qd, 1), lambda i: (0, 0)),
            pl.BlockSpec((1, att_dim), lambda i: (0, 0)),
            pl.BlockSpec((1, att_dim), lambda i: (0, 0)),
        ],
        out_specs=pl.BlockSpec((BB2, att_dim), lambda i: (i, 0)),
        out_shape=jax.ShapeDtypeStruct((B, att_dim), f32),
    )(yg, ka, bgt, g3t, b3t, gsel, Wa, bar, qr, g4r, b4r)
    return out
